# Initial kernel scaffold; baseline (speedup 1.0000x reference)
import jax
import jax.numpy as jnp
from jax.experimental import pallas as pl


def kernel(token_ids, omega_table, pi_table):
    raise NotImplementedError("write your pallas kernel here")


# speedup vs baseline: 5.9506x; 5.9506x over previous
import functools
import math

import jax
import jax.numpy as jnp
from jax import lax
from jax.experimental import pallas as pl
from jax.experimental.pallas import tpu as pltpu
from jax.experimental.pallas import tpu_sc as plsc

VOCAB = 100000
DIM = 64
D2 = DIM * 2
D4 = DIM * 4
MAX_SEQ = 512
B = 1024
S = 200
N = B * S
NW = 32
CH = 100
G = S
PER_W = N // NW
NG = PER_W // G
NCH = PER_W // CH
LANES = 16


def _pos_table():
    position = jnp.arange(MAX_SEQ, dtype=jnp.float32)[:, None]
    div_term = jnp.exp(
        jnp.arange(0, DIM, 2).astype(jnp.float32) * (-math.log(10000.0) / DIM)
    )
    pe_sin = jnp.sin(position * div_term)
    pe_cos = jnp.cos(position * div_term)
    pe_real = jnp.zeros((MAX_SEQ, DIM), jnp.float32)
    pe_real = pe_real.at[:, 0::2].set(pe_sin)
    pe_real = pe_real.at[:, 1::2].set(pe_cos)
    pe_imag = jnp.zeros((MAX_SEQ, DIM), jnp.float32)
    pe_imag = pe_imag.at[:, 0::2].set(pe_cos)
    pe_imag = pe_imag.at[:, 1::2].set(-pe_sin)
    return jnp.concatenate([pe_real, pe_imag], axis=-1)[:S]


def _sc_embed(tok2d, omega_table, pi_table, pos):
    mesh = plsc.VectorSubcoreMesh(core_axis_name="c", subcore_axis_name="s")

    @functools.partial(
        pl.kernel,
        out_type=jax.ShapeDtypeStruct((N, D4), jnp.float32),
        mesh=mesh,
        scratch_types=[
            pltpu.VMEM((NCH, CH), jnp.int32),
            pltpu.VMEM((S, D2), jnp.float32),
            pltpu.VMEM((G, D2), jnp.float32),
            pltpu.VMEM((G, D2), jnp.float32),
            pltpu.SemaphoreType.DMA,
            pltpu.SemaphoreType.DMA,
        ],
    )
    def k(tok_hbm, omega_hbm, pi_hbm, pos_hbm, out_hbm,
          idx_v, pos_v, om_v, pi_v, sem_o, sem_p):
        wid = lax.axis_index("s") * 2 + lax.axis_index("c")
        pltpu.sync_copy(pos_hbm, pos_v)
        pltpu.sync_copy(tok_hbm.at[pl.ds(wid * NCH, NCH)], idx_v)

        def group_body(g, carry):
            c0 = 2 * g
            cps = [
                pltpu.async_copy(omega_hbm.at[idx_v.at[c0]],
                                 om_v.at[pl.ds(0, CH)], sem_o),
                pltpu.async_copy(omega_hbm.at[idx_v.at[c0 + 1]],
                                 om_v.at[pl.ds(CH, CH)], sem_o),
                pltpu.async_copy(pi_hbm.at[idx_v.at[c0]],
                                 pi_v.at[pl.ds(0, CH)], sem_p),
                pltpu.async_copy(pi_hbm.at[idx_v.at[c0 + 1]],
                                 pi_v.at[pl.ds(CH, CH)], sem_p),
            ]
            for cp in cps:
                cp.wait()

            def row_body(j, carry2):
                for h in range(D2 // LANES):
                    sl = pl.ds(h * LANES, LANES)
                    p = pos_v[j, sl]
                    om_v[j, sl] = om_v[j, sl] + p
                    pi_v[j, sl] = pi_v[j, sl] + p
                return carry2

            lax.fori_loop(0, G, row_body, 0)
            r0 = wid * PER_W + g * G
            pltpu.sync_copy(om_v, out_hbm.at[pl.ds(r0, G), pl.ds(0, D2)])
            pltpu.sync_copy(pi_v, out_hbm.at[pl.ds(r0, G), pl.ds(D2, D2)])
            return carry

        lax.fori_loop(0, NG, group_body, 0)

    return k(tok2d, omega_table, pi_table, pos)


def kernel(token_ids, omega_table, pi_table):
    tok2d = token_ids.reshape(N // CH, CH).astype(jnp.int32)
    pos = _pos_table()
    out = _sc_embed(tok2d, omega_table, pi_table, pos)
    return out.reshape(B, S, D4)
